# baseline (speedup 1.0000x reference)
import dataclasses
import functools

import jax
import jax.numpy as jnp
from jax import lax
from jax.experimental import pallas as pl
from jax.experimental.pallas import tpu as pltpu
from jax.experimental.pallas import tpu_sc as plsc

N = 10000
E = 320000
D = 128
OUT = 128

NC = 2
NS = 16
NW = NC * NS
CH = 128
NCHUNKS = E // CH
ROWS_PER_SUB = 624
TAIL_ROWS = N - NS * ROWS_PER_SUB
LANES = 16


def _sc_aggregate(x, src2, dst2, wt2, zeros):
    mesh = plsc.VectorSubcoreMesh(core_axis_name="c", subcore_axis_name="s")
    cp = pltpu.CompilerParams()
    if "needs_layout_passes" in pltpu.CompilerParams.__dataclass_fields__:
        cp = dataclasses.replace(cp, needs_layout_passes=False)

    @functools.partial(
        pl.kernel,
        out_type=jax.ShapeDtypeStruct((NC, N, D), jnp.float32),
        mesh=mesh,
        compiler_params=cp,
        scratch_types=[
            pltpu.VMEM((CH,), jnp.int32),
            pltpu.VMEM((CH,), jnp.int32),
            pltpu.VMEM((CH,), jnp.float32),
            pltpu.VMEM((CH,), jnp.int32),
            pltpu.VMEM((CH,), jnp.int32),
            pltpu.VMEM((CH,), jnp.float32),
            pltpu.VMEM((CH, D), jnp.float32),
            pltpu.VMEM((CH, D), jnp.float32),
            pltpu.VMEM_SHARED((N, D), jnp.float32),
            pltpu.SemaphoreType.DMA,
            pltpu.SemaphoreType.DMA,
            pltpu.SemaphoreType.DMA,
            pltpu.SemaphoreType.DMA,
            pltpu.SemaphoreType.DMA,
            pltpu.SemaphoreType.DMA,
        ],
    )
    def agg(x_hbm, src_hbm, dst_hbm, wt_hbm, z_hbm, hp_hbm,
            src_a, dst_a, wt_a, src_b, dst_b, wt_b, rows_a, rows_b,
            h_sh, gsem_a, gsem_b, ssem_a, ssem_b, isem_a, isem_b):
        cid = lax.axis_index("c")
        sid = lax.axis_index("s")
        wid = sid * NC + cid
        row0 = sid * ROWS_PER_SUB

        pltpu.sync_copy(z_hbm.at[pl.ds(row0, ROWS_PER_SUB)],
                        h_sh.at[pl.ds(row0, ROWS_PER_SUB)])

        @pl.when(sid == NS - 1)
        def _zero_tail():
            pltpu.sync_copy(z_hbm.at[pl.ds(NS * ROWS_PER_SUB, TAIL_ROWS)],
                            h_sh.at[pl.ds(NS * ROWS_PER_SUB, TAIL_ROWS)])

        plsc.subcore_barrier()

        def scale_rows(rows_v, wt_v):
            @pl.loop(0, CH, step=8)
            def _edge(e0):
                for jj in range(8):
                    e = e0 + jj
                    w = plsc.load_gather(
                        wt_v, [jnp.full((LANES,), e, jnp.int32)])
                    for dd in range(D // LANES):
                        sl = pl.ds(dd * LANES, LANES)
                        rows_v[e, sl] = rows_v[e, sl] * w

        pltpu.sync_copy(src_hbm.at[wid], src_a)
        pltpu.async_copy(x_hbm.at[src_a], rows_a, gsem_a)
        pltpu.async_copy(dst_hbm.at[wid], dst_a, isem_a)
        pltpu.async_copy(wt_hbm.at[wid], wt_a, isem_a)

        @pl.loop(wid, NCHUNKS - NW, step=2 * NW)
        def _pair(c):
            cb = c + NW

            @pl.when(c != wid)
            def _drain_prev():
                pltpu.make_async_copy(rows_b, h_sh.at[dst_b], ssem_b).wait()

            pltpu.sync_copy(src_hbm.at[cb], src_b)
            pltpu.async_copy(x_hbm.at[src_b], rows_b, gsem_b)
            pltpu.async_copy(dst_hbm.at[cb], dst_b, isem_b)
            pltpu.async_copy(wt_hbm.at[cb], wt_b, isem_b)

            pltpu.make_async_copy(x_hbm.at[src_a], rows_a, gsem_a).wait()
            pltpu.make_async_copy(dst_hbm.at[c], dst_a, isem_a).wait()
            pltpu.make_async_copy(wt_hbm.at[c], wt_a, isem_a).wait()
            scale_rows(rows_a, wt_a)
            pltpu.async_copy(rows_a, h_sh.at[dst_a], ssem_a, add=True)

            pltpu.make_async_copy(x_hbm.at[src_b], rows_b, gsem_b).wait()
            pltpu.make_async_copy(dst_hbm.at[cb], dst_b, isem_b).wait()
            pltpu.make_async_copy(wt_hbm.at[cb], wt_b, isem_b).wait()
            scale_rows(rows_b, wt_b)
            pltpu.make_async_copy(rows_a, h_sh.at[dst_a], ssem_a).wait()

            @pl.when(c + 2 * NW < NCHUNKS - NW)
            def _prefetch_next():
                ca = c + 2 * NW
                pltpu.sync_copy(src_hbm.at[ca], src_a)
                pltpu.async_copy(x_hbm.at[src_a], rows_a, gsem_a)
                pltpu.async_copy(dst_hbm.at[ca], dst_a, isem_a)
                pltpu.async_copy(wt_hbm.at[ca], wt_a, isem_a)

            pltpu.async_copy(rows_b, h_sh.at[dst_b], ssem_b, add=True)

        pltpu.make_async_copy(rows_b, h_sh.at[dst_b], ssem_b).wait()

        @pl.when(wid < (NCHUNKS - NW * (NCHUNKS // NW)))
        def _tail_chunk():
            c = NW * (NCHUNKS // NW) + wid
            pltpu.sync_copy(src_hbm.at[c], src_a)
            pltpu.sync_copy(dst_hbm.at[c], dst_a)
            pltpu.sync_copy(wt_hbm.at[c], wt_a)
            pltpu.async_copy(x_hbm.at[src_a], rows_a, gsem_a).wait()
            scale_rows(rows_a, wt_a)
            pltpu.sync_copy(rows_a, h_sh.at[dst_a], add=True)

        plsc.subcore_barrier()
        pltpu.sync_copy(h_sh.at[pl.ds(row0, ROWS_PER_SUB)],
                        hp_hbm.at[cid, pl.ds(row0, ROWS_PER_SUB)])

        @pl.when(sid == NS - 1)
        def _flush_tail():
            pltpu.sync_copy(h_sh.at[pl.ds(NS * ROWS_PER_SUB, TAIL_ROWS)],
                            hp_hbm.at[cid, pl.ds(NS * ROWS_PER_SUB, TAIL_ROWS)])

    return agg(x, src2, dst2, wt2, zeros)


def _tc_project(h0, h1, x, wt, wb):
    RB = 1000

    def body(h0_ref, h1_ref, x_ref, wt_ref, wb_ref, o_ref):
        h = h0_ref[...] + h1_ref[...]
        acc = jnp.dot(h, wt_ref[...], preferred_element_type=jnp.float32)
        acc = acc + jnp.dot(x_ref[...], wb_ref[...],
                            preferred_element_type=jnp.float32)
        o_ref[...] = jnp.maximum(acc, 0.0)

    return pl.pallas_call(
        body,
        grid=(N // RB,),
        in_specs=[
            pl.BlockSpec((RB, D), lambda i: (i, 0)),
            pl.BlockSpec((RB, D), lambda i: (i, 0)),
            pl.BlockSpec((RB, D), lambda i: (i, 0)),
            pl.BlockSpec((D, OUT), lambda i: (0, 0)),
            pl.BlockSpec((D, OUT), lambda i: (0, 0)),
        ],
        out_specs=pl.BlockSpec((RB, OUT), lambda i: (i, 0)),
        out_shape=jax.ShapeDtypeStruct((N, OUT), jnp.float32),
    )(h0, h1, x, wt, wb)


def kernel(x, edge_index, edge_weight, W):
    src2 = edge_index[1].reshape(NCHUNKS, CH)
    dst2 = edge_index[0].reshape(NCHUNKS, CH)
    wt2 = edge_weight.reshape(NCHUNKS, CH)
    zeros = jnp.zeros((N, D), jnp.float32)
    hp = _sc_aggregate(x, src2, dst2, wt2, zeros)
    return _tc_project(hp[0], hp[1], x, W[:D], W[D:])
